# baseline (speedup 1.0000x reference)
import functools

import jax
import jax.numpy as jnp
from jax import lax
from jax.experimental import pallas as pl
from jax.experimental.pallas import tpu as pltpu
from jax.experimental.pallas import tpu_sc as plsc

B = 16384
NC = 26
NN = 13
V = 100000
D = 32
H = 32
NT = NC + NN + 1

NW = 32
CB = 4096
NCH = B // CB


def _sc_gather(table_t3, idx_t, pos_cat):
    mesh = plsc.VectorSubcoreMesh(core_axis_name="c", subcore_axis_name="s")

    @functools.partial(
        pl.kernel,
        mesh=mesh,
        out_type=jax.ShapeDtypeStruct((NT, D, B), jnp.float32),
        scratch_types=(
            [pltpu.VMEM((V,), jnp.float32)]
            + [pltpu.VMEM((CB,), jnp.int32) for _ in range(2)]
            + [pltpu.VMEM((CB,), jnp.float32) for _ in range(2)]
            + [pltpu.VMEM((NC * D,), jnp.float32)]
            + [pltpu.SemaphoreType.DMA, pltpu.SemaphoreType.DMA,
               pltpu.SemaphoreType.DMA]
        ),
        compiler_params=pltpu.CompilerParams(use_tc_tiling_on_sc=True,
                                             needs_layout_passes=False),
    )
    def k(table_hbm, idx_hbm, pos_hbm, out_hbm, row_v, ib0, ib1, ob0, ob1,
          pos_v, isem, wsem, rsem):
        d = lax.axis_index("s") * 2 + lax.axis_index("c")
        ibufs = [ib0, ib1]
        obufs = [ob0, ob1]
        pltpu.sync_copy(pos_hbm, pos_v)

        def drain_two_writes():
            pltpu.make_async_copy(ob0, out_hbm.at[0, 0, pl.ds(0, CB)],
                                  wsem).wait()
            pltpu.make_async_copy(ob1, out_hbm.at[0, 0, pl.ds(0, CB)],
                                  wsem).wait()

        def col_body(c, carry):
            rdesc = pltpu.async_copy(table_hbm.at[c, d, :], row_v, rsem)
            idescs = [pltpu.async_copy(idx_hbm.at[c, pl.ds(0, CB)], ib0,
                                       isem), None]
            pidx = jnp.zeros((16,), jnp.int32) + (c * D + d)
            pv = plsc.load_gather(pos_v, [pidx])

            @pl.when(c > 0)
            def _():
                drain_two_writes()

            wdescs = [None, None]
            for ch in range(NCH):
                q = ch % 2
                if ch + 1 < NCH:
                    idescs[1 - q] = pltpu.async_copy(
                        idx_hbm.at[c, pl.ds((ch + 1) * CB, CB)],
                        ibufs[1 - q], isem)
                idescs[q].wait()
                if ch == 0:
                    rdesc.wait()
                if wdescs[q] is not None:
                    wdescs[q].wait()
                ib = ibufs[q]
                ob = obufs[q]

                def vec_body(j, car):
                    for i in range(8):
                        o = (j * 8 + i) * 16
                        vidx = ib[pl.ds(o, 16)]
                        ob[pl.ds(o, 16)] = plsc.load_gather(row_v,
                                                            [vidx]) + pv
                    return car

                lax.fori_loop(0, CB // 128, vec_body, 0)
                wdescs[q] = pltpu.async_copy(
                    ob, out_hbm.at[1 + c, d, pl.ds(ch * CB, CB)], wsem)
            return carry

        lax.fori_loop(0, NC, col_body, 0)
        drain_two_writes()

    return k(table_t3, idx_t, pos_cat)


def _tc_cls_body(buf_ref, cls_ref, pos_ref, out_ref):
    bb = out_ref.shape[2]
    out_ref[0, :, :] = jnp.broadcast_to(cls_ref[0] + pos_ref[0], (D, bb))


def _tc_cls(buf, clst, post, interpret=False):
    BB = 2048
    return pl.pallas_call(
        _tc_cls_body,
        grid=(B // BB,),
        in_specs=[
            pl.BlockSpec(memory_space=pltpu.MemorySpace.HBM),
            pl.BlockSpec((1, D, 1), lambda i: (0, 0, 0)),
            pl.BlockSpec((1, D, 1), lambda i: (0, 0, 0)),
        ],
        out_specs=pl.BlockSpec((1, D, BB), lambda i: (0, 0, i)),
        out_shape=jax.ShapeDtypeStruct((NT, D, B), jnp.float32),
        input_output_aliases={0: 0},
        interpret=interpret,
    )(buf, clst, post)


def _tc_num_body(buf_ref, nv_ref, mf_ref, nf_ref, w1_ref, b1_ref, w2_ref,
                 b2_ref, me_ref, ne_ref, pos_ref, out_ref):
    vr = nv_ref[0]
    mfr = mf_ref[0]
    nfr = nf_ref[0]
    sp = jnp.maximum(mfr, nfr)
    v0 = vr * (1.0 - sp)
    h = jnp.maximum(w1_ref[0] * v0 + b1_ref[0], 0.0)
    o = jnp.dot(w2_ref[0], h, preferred_element_type=jnp.float32)
    o = o + b2_ref[0]
    o = jnp.where(mfr > 0.5, me_ref[0], o)
    o = jnp.where(nfr > 0.5, ne_ref[0], o)
    out_ref[0, :, :] = o + pos_ref[0]


def _tc_num(buf, nv_t, mf_t, nf_t, w1t, b1t, w2t, b2t, met, net, post,
            interpret=False):
    BB = 2048
    return pl.pallas_call(
        _tc_num_body,
        grid=(NN, B // BB),
        in_specs=[
            pl.BlockSpec(memory_space=pltpu.MemorySpace.HBM),
            pl.BlockSpec((1, 1, BB), lambda n, i: (n, 0, i)),
            pl.BlockSpec((1, 1, BB), lambda n, i: (n, 0, i)),
            pl.BlockSpec((1, 1, BB), lambda n, i: (n, 0, i)),
            pl.BlockSpec((1, D, 1), lambda n, i: (n, 0, 0)),
            pl.BlockSpec((1, D, 1), lambda n, i: (n, 0, 0)),
            pl.BlockSpec((1, D, D), lambda n, i: (n, 0, 0)),
            pl.BlockSpec((1, D, 1), lambda n, i: (n, 0, 0)),
            pl.BlockSpec((1, D, 1), lambda n, i: (n, 0, 0)),
            pl.BlockSpec((1, D, 1), lambda n, i: (n, 0, 0)),
            pl.BlockSpec((1, D, 1), lambda n, i: (n, 0, 0)),
        ],
        out_specs=pl.BlockSpec((1, D, BB), lambda n, i: (1 + NC + n, 0, i)),
        out_shape=jax.ShapeDtypeStruct((NT, D, B), jnp.float32),
        input_output_aliases={0: 0},
        interpret=interpret,
    )(buf, nv_t, mf_t, nf_t, w1t, b1t, w2t, b2t, met, net, post)


def kernel(cat_indices, numeric_values, mask_flags, null_flags, emb_tables,
           W1, b1, W2, b2, mask_emb, null_emb, cls_token, pos_table):
    table_t3 = jnp.transpose(emb_tables, (0, 2, 1))
    idx_t = cat_indices.astype(jnp.int32).T
    pos_cat = pos_table[1:1 + NC].reshape(NC * D)
    buf = _sc_gather(table_t3, idx_t, pos_cat)

    nv_t = numeric_values.T[:, None, :]
    mf_t = mask_flags.T.astype(jnp.float32)[:, None, :]
    nf_t = null_flags.T.astype(jnp.float32)[:, None, :]
    w1t = W1.reshape(NN, H)[:, :, None]
    b1t = b1[:, :, None]
    w2t = jnp.transpose(W2, (0, 2, 1))
    b2t = b2[:, :, None]
    met = mask_emb[:, :, None]
    net = null_emb[:, :, None]
    clst = cls_token.reshape(1, D, 1)
    pos0 = pos_table[0].reshape(1, D, 1)
    posn = pos_table[1 + NC:][:, :, None]

    buf = _tc_cls(buf, clst, pos0)
    buf = _tc_num(buf, nv_t, mf_t, nf_t, w1t, b1t, w2t, b2t, met, net, posn)
    return jnp.transpose(buf, (2, 0, 1))


# speedup vs baseline: 1.1430x; 1.1430x over previous
import functools

import jax
import jax.numpy as jnp
from jax import lax
from jax.experimental import pallas as pl
from jax.experimental.pallas import tpu as pltpu
from jax.experimental.pallas import tpu_sc as plsc

B = 16384
NC = 26
NN = 13
V = 100000
D = 32
H = 32
NT = NC + NN + 1

NW = 32
CB = 4096
NCH = B // CB


def _sc_gather(table_t3, idx_t, pos_cat):
    mesh = plsc.VectorSubcoreMesh(core_axis_name="c", subcore_axis_name="s")

    @functools.partial(
        pl.kernel,
        mesh=mesh,
        out_type=jax.ShapeDtypeStruct((NT, D, B), jnp.float32),
        scratch_types=(
            [pltpu.VMEM((V,), jnp.float32)]
            + [pltpu.VMEM((CB,), jnp.int32) for _ in range(2)]
            + [pltpu.VMEM((CB,), jnp.float32) for _ in range(2)]
            + [pltpu.VMEM((NC * D,), jnp.float32)]
            + [pltpu.SemaphoreType.DMA, pltpu.SemaphoreType.DMA,
               pltpu.SemaphoreType.DMA]
        ),
        compiler_params=pltpu.CompilerParams(use_tc_tiling_on_sc=True,
                                             needs_layout_passes=False),
    )
    def k(table_hbm, idx_hbm, pos_hbm, out_hbm, row_v, ib0, ib1, ob0, ob1,
          pos_v, isem, wsem, rsem):
        d = lax.axis_index("s") * 2 + lax.axis_index("c")
        ibufs = [ib0, ib1]
        obufs = [ob0, ob1]
        pltpu.sync_copy(pos_hbm, pos_v)

        def drain_two_writes():
            pltpu.make_async_copy(ob0, out_hbm.at[0, 0, pl.ds(0, CB)],
                                  wsem).wait()
            pltpu.make_async_copy(ob1, out_hbm.at[0, 0, pl.ds(0, CB)],
                                  wsem).wait()

        def col_body(c, carry):
            rdesc = pltpu.async_copy(table_hbm.at[c, d, :], row_v, rsem)
            idescs = [pltpu.async_copy(idx_hbm.at[c, pl.ds(0, CB)], ib0,
                                       isem), None]
            pidx = jnp.zeros((16,), jnp.int32) + (c * D + d)
            pv = plsc.load_gather(pos_v, [pidx])

            @pl.when(c > 0)
            def _():
                drain_two_writes()

            wdescs = [None, None]
            for ch in range(NCH):
                q = ch % 2
                if ch + 1 < NCH:
                    idescs[1 - q] = pltpu.async_copy(
                        idx_hbm.at[c, pl.ds((ch + 1) * CB, CB)],
                        ibufs[1 - q], isem)
                idescs[q].wait()
                if ch == 0:
                    rdesc.wait()
                if wdescs[q] is not None:
                    wdescs[q].wait()
                ib = ibufs[q]
                ob = obufs[q]

                def vec_body(j, car):
                    for i in range(8):
                        o = (j * 8 + i) * 16
                        vidx = ib[pl.ds(o, 16)]
                        ob[pl.ds(o, 16)] = plsc.load_gather(row_v,
                                                            [vidx]) + pv
                    return car

                lax.fori_loop(0, CB // 128, vec_body, 0)
                wdescs[q] = pltpu.async_copy(
                    ob, out_hbm.at[1 + c, d, pl.ds(ch * CB, CB)], wsem)
            return carry

        lax.fori_loop(0, NC, col_body, 0)
        drain_two_writes()

    return k(table_t3, idx_t, pos_cat)


def _tc_cls_body(buf_ref, cls_ref, pos_ref, out_ref):
    bb = out_ref.shape[2]
    out_ref[0, :, :] = jnp.broadcast_to(cls_ref[0] + pos_ref[0], (D, bb))


def _tc_cls(buf, clst, post, interpret=False):
    BB = B
    return pl.pallas_call(
        _tc_cls_body,
        grid=(B // BB,),
        in_specs=[
            pl.BlockSpec(memory_space=pltpu.MemorySpace.HBM),
            pl.BlockSpec((1, D, 1), lambda i: (0, 0, 0)),
            pl.BlockSpec((1, D, 1), lambda i: (0, 0, 0)),
        ],
        out_specs=pl.BlockSpec((1, D, BB), lambda i: (0, 0, i)),
        out_shape=jax.ShapeDtypeStruct((NT, D, B), jnp.float32),
        input_output_aliases={0: 0},
        interpret=interpret,
    )(buf, clst, post)


def _tc_num_body(buf_ref, nv_ref, mf_ref, nf_ref, w1_ref, b1_ref, w2_ref,
                 b2_ref, me_ref, ne_ref, pos_ref, out_ref):
    vr = nv_ref[0]
    mfr = mf_ref[0]
    nfr = nf_ref[0]
    sp = jnp.maximum(mfr, nfr)
    v0 = vr * (1.0 - sp)
    h = jnp.maximum(w1_ref[0] * v0 + b1_ref[0], 0.0)
    o = jnp.dot(w2_ref[0], h, preferred_element_type=jnp.float32)
    o = o + b2_ref[0]
    o = jnp.where(mfr > 0.5, me_ref[0], o)
    o = jnp.where(nfr > 0.5, ne_ref[0], o)
    out_ref[0, :, :] = o + pos_ref[0]


def _tc_num(buf, nv_t, mf_t, nf_t, w1t, b1t, w2t, b2t, met, net, post,
            interpret=False):
    BB = B
    return pl.pallas_call(
        _tc_num_body,
        grid=(NN,),
        in_specs=[
            pl.BlockSpec(memory_space=pltpu.MemorySpace.HBM),
            pl.BlockSpec((1, 1, BB), lambda n: (n, 0, 0)),
            pl.BlockSpec((1, 1, BB), lambda n: (n, 0, 0)),
            pl.BlockSpec((1, 1, BB), lambda n: (n, 0, 0)),
            pl.BlockSpec((1, D, 1), lambda n: (n, 0, 0)),
            pl.BlockSpec((1, D, 1), lambda n: (n, 0, 0)),
            pl.BlockSpec((1, D, D), lambda n: (n, 0, 0)),
            pl.BlockSpec((1, D, 1), lambda n: (n, 0, 0)),
            pl.BlockSpec((1, D, 1), lambda n: (n, 0, 0)),
            pl.BlockSpec((1, D, 1), lambda n: (n, 0, 0)),
            pl.BlockSpec((1, D, 1), lambda n: (n, 0, 0)),
        ],
        out_specs=pl.BlockSpec((1, D, BB), lambda n: (1 + NC + n, 0, 0)),
        out_shape=jax.ShapeDtypeStruct((NT, D, B), jnp.float32),
        input_output_aliases={0: 0},
        interpret=interpret,
    )(buf, nv_t, mf_t, nf_t, w1t, b1t, w2t, b2t, met, net, post)


def kernel(cat_indices, numeric_values, mask_flags, null_flags, emb_tables,
           W1, b1, W2, b2, mask_emb, null_emb, cls_token, pos_table):
    table_t3 = jnp.transpose(emb_tables, (0, 2, 1))
    idx_t = cat_indices.astype(jnp.int32).T
    pos_cat = pos_table[1:1 + NC].reshape(NC * D)
    buf = _sc_gather(table_t3, idx_t, pos_cat)

    nv_t = numeric_values.T[:, None, :]
    mf_t = mask_flags.T.astype(jnp.float32)[:, None, :]
    nf_t = null_flags.T.astype(jnp.float32)[:, None, :]
    w1t = W1.reshape(NN, H)[:, :, None]
    b1t = b1[:, :, None]
    w2t = jnp.transpose(W2, (0, 2, 1))
    b2t = b2[:, :, None]
    met = mask_emb[:, :, None]
    net = null_emb[:, :, None]
    clst = cls_token.reshape(1, D, 1)
    pos0 = pos_table[0].reshape(1, D, 1)
    posn = pos_table[1 + NC:][:, :, None]

    buf = _tc_cls(buf, clst, pos0)
    buf = _tc_num(buf, nv_t, mf_t, nf_t, w1t, b1t, w2t, b2t, met, net, posn)
    return jnp.transpose(buf, (2, 0, 1))


# speedup vs baseline: 1.1949x; 1.0454x over previous
import functools

import jax
import jax.numpy as jnp
from jax import lax
from jax.experimental import pallas as pl
from jax.experimental.pallas import tpu as pltpu
from jax.experimental.pallas import tpu_sc as plsc

B = 16384
NC = 26
NN = 13
V = 100000
D = 32
H = 32
NT = NC + NN + 1

NW = 32
CB = 4096
NCH = B // CB


def _sc_gather(table_t3, idx_t):
    mesh = plsc.VectorSubcoreMesh(core_axis_name="c", subcore_axis_name="s")

    @functools.partial(
        pl.kernel,
        mesh=mesh,
        out_type=jax.ShapeDtypeStruct((NC, D, B), jnp.float32),
        scratch_types=(
            [pltpu.VMEM((V,), jnp.float32)]
            + [pltpu.VMEM((CB,), jnp.int32) for _ in range(2)]
            + [pltpu.VMEM((CB,), jnp.float32) for _ in range(2)]
            + [pltpu.SemaphoreType.DMA, pltpu.SemaphoreType.DMA,
               pltpu.SemaphoreType.DMA]
        ),
        compiler_params=pltpu.CompilerParams(use_tc_tiling_on_sc=True,
                                             needs_layout_passes=False),
    )
    def k(table_hbm, idx_hbm, out_hbm, row_v, ib0, ib1, ob0, ob1, isem, wsem,
          rsem):
        d = lax.axis_index("s") * 2 + lax.axis_index("c")
        ibufs = [ib0, ib1]
        obufs = [ob0, ob1]

        def drain_two_writes():
            pltpu.make_async_copy(ob0, out_hbm.at[0, 0, pl.ds(0, CB)],
                                  wsem).wait()
            pltpu.make_async_copy(ob1, out_hbm.at[0, 0, pl.ds(0, CB)],
                                  wsem).wait()

        def col_body(c, carry):
            rdescs = [pltpu.async_copy(table_hbm.at[c, d, :], row_v, rsem)]
            idescs = [pltpu.async_copy(idx_hbm.at[c, pl.ds(0, CB)], ib0,
                                       isem), None]

            @pl.when(c > 0)
            def _():
                drain_two_writes()

            wdescs = [None, None]
            for ch in range(NCH):
                q = ch % 2
                if ch + 1 < NCH:
                    idescs[1 - q] = pltpu.async_copy(
                        idx_hbm.at[c, pl.ds((ch + 1) * CB, CB)],
                        ibufs[1 - q], isem)
                idescs[q].wait()
                if ch == 0:
                    for rd in rdescs:
                        rd.wait()
                if wdescs[q] is not None:
                    wdescs[q].wait()
                ib = ibufs[q]
                ob = obufs[q]

                def vec_body(j, car):
                    for i in range(8):
                        o = (j * 8 + i) * 16
                        vidx = ib[pl.ds(o, 16)]
                        ob[pl.ds(o, 16)] = plsc.load_gather(row_v, [vidx])
                    return car

                lax.fori_loop(0, CB // 128, vec_body, 0)
                wdescs[q] = pltpu.async_copy(
                    ob, out_hbm.at[c, d, pl.ds(ch * CB, CB)], wsem)
            return carry

        lax.fori_loop(0, NC, col_body, 0)
        drain_two_writes()

    return k(table_t3, idx_t)


def _tc_assemble_body(cat_ref, nv_ref, mf_ref, nf_ref, w1_ref, b1_ref,
                      w2_ref, b2_ref, me_ref, ne_ref, cls_ref, pos_ref,
                      posc_ref, out_ref):
    bb = out_ref.shape[2]
    out_ref[0, :, :] = jnp.broadcast_to(cls_ref[...] + pos_ref[:, 0:1],
                                        (D, bb))
    out_ref[1:1 + NC, :, :] = cat_ref[...] + posc_ref[...]
    for n in range(NN):
        vr = nv_ref[n:n + 1, :]
        mfr = mf_ref[n:n + 1, :]
        nfr = nf_ref[n:n + 1, :]
        sp = jnp.maximum(mfr, nfr)
        v0 = vr * (1.0 - sp)
        h = jnp.maximum(w1_ref[:, n:n + 1] * v0 + b1_ref[:, n:n + 1], 0.0)
        o = jnp.dot(w2_ref[n], h, preferred_element_type=jnp.float32)
        o = o + b2_ref[:, n:n + 1]
        o = jnp.where(mfr > 0.5, me_ref[:, n:n + 1], o)
        o = jnp.where(nfr > 0.5, ne_ref[:, n:n + 1], o)
        out_ref[1 + NC + n, :, :] = o + pos_ref[:, 1 + NC + n:2 + NC + n]


def _tc_assemble(cat_t, nv_t, mf_t, nf_t, w1t, b1t, w2t, b2t, met, net,
                 clst, post, posc3, interpret=False):
    BB = 2048
    grid = (B // BB,)
    return pl.pallas_call(
        _tc_assemble_body,
        grid=grid,
        in_specs=[
            pl.BlockSpec((NC, D, BB), lambda i: (0, 0, i)),
            pl.BlockSpec((NN, BB), lambda i: (0, i)),
            pl.BlockSpec((NN, BB), lambda i: (0, i)),
            pl.BlockSpec((NN, BB), lambda i: (0, i)),
            pl.BlockSpec((D, NN), lambda i: (0, 0)),
            pl.BlockSpec((D, NN), lambda i: (0, 0)),
            pl.BlockSpec((NN, D, D), lambda i: (0, 0, 0)),
            pl.BlockSpec((D, NN), lambda i: (0, 0)),
            pl.BlockSpec((D, NN), lambda i: (0, 0)),
            pl.BlockSpec((D, NN), lambda i: (0, 0)),
            pl.BlockSpec((D, 1), lambda i: (0, 0)),
            pl.BlockSpec((D, NT), lambda i: (0, 0)),
            pl.BlockSpec((NC, D, 1), lambda i: (0, 0, 0)),
        ],
        out_specs=pl.BlockSpec((NT, D, BB), lambda i: (0, 0, i)),
        out_shape=jax.ShapeDtypeStruct((NT, D, B), jnp.float32),
        interpret=interpret,
    )(cat_t, nv_t, mf_t, nf_t, w1t, b1t, w2t, b2t, met, net, clst, post,
      posc3)


def kernel(cat_indices, numeric_values, mask_flags, null_flags, emb_tables,
           W1, b1, W2, b2, mask_emb, null_emb, cls_token, pos_table):
    table_t3 = jnp.transpose(emb_tables, (0, 2, 1))
    idx_t = cat_indices.astype(jnp.int32).T
    cat_t = _sc_gather(table_t3, idx_t)

    nv_t = numeric_values.T
    mf_t = mask_flags.T.astype(jnp.float32)
    nf_t = null_flags.T.astype(jnp.float32)
    w1t = W1.reshape(NN, H).T
    b1t = b1.T
    w2t = jnp.transpose(W2, (0, 2, 1))
    b2t = b2.T
    met = mask_emb.T
    net = null_emb.T
    clst = cls_token.reshape(1, D).T
    post = pos_table.T
    posc3 = pos_table[1:1 + NC][:, :, None]

    out_t = _tc_assemble(cat_t, nv_t, mf_t, nf_t, w1t, b1t, w2t, b2t,
                         met, net, clst, post, posc3)
    return jnp.transpose(out_t, (2, 0, 1))


# speedup vs baseline: 1.2020x; 1.0060x over previous
import functools

import jax
import jax.numpy as jnp
from jax import lax
from jax.experimental import pallas as pl
from jax.experimental.pallas import tpu as pltpu
from jax.experimental.pallas import tpu_sc as plsc

B = 16384
NC = 26
NN = 13
V = 100000
D = 32
H = 32
NT = NC + NN + 1

NW = 32
CB = 4096
NCH = B // CB


def _sc_gather(table_t3, idx_t):
    mesh = plsc.VectorSubcoreMesh(core_axis_name="c", subcore_axis_name="s")

    @functools.partial(
        pl.kernel,
        mesh=mesh,
        out_type=jax.ShapeDtypeStruct((NC, D, B), jnp.float32),
        scratch_types=(
            [pltpu.VMEM((V,), jnp.float32)]
            + [pltpu.VMEM((CB,), jnp.int32) for _ in range(2)]
            + [pltpu.VMEM((CB,), jnp.float32) for _ in range(2)]
            + [pltpu.SemaphoreType.DMA, pltpu.SemaphoreType.DMA,
               pltpu.SemaphoreType.DMA]
        ),
        compiler_params=pltpu.CompilerParams(use_tc_tiling_on_sc=True,
                                             needs_layout_passes=False),
    )
    def k(table_hbm, idx_hbm, out_hbm, row_v, ib0, ib1, ob0, ob1, isem, wsem,
          rsem):
        d = lax.axis_index("s") * 2 + lax.axis_index("c")
        ibufs = [ib0, ib1]
        obufs = [ob0, ob1]

        def drain_two_writes():
            pltpu.make_async_copy(ob0, out_hbm.at[0, 0, pl.ds(0, CB)],
                                  wsem).wait()
            pltpu.make_async_copy(ob1, out_hbm.at[0, 0, pl.ds(0, CB)],
                                  wsem).wait()

        def col_body(c, carry):
            rdescs = [pltpu.async_copy(table_hbm.at[c, d, :], row_v, rsem)]
            idescs = [pltpu.async_copy(idx_hbm.at[c, pl.ds(0, CB)], ib0,
                                       isem), None]

            @pl.when(c > 0)
            def _():
                drain_two_writes()

            wdescs = [None, None]
            for ch in range(NCH):
                q = ch % 2
                if ch + 1 < NCH:
                    idescs[1 - q] = pltpu.async_copy(
                        idx_hbm.at[c, pl.ds((ch + 1) * CB, CB)],
                        ibufs[1 - q], isem)
                idescs[q].wait()
                if ch == 0:
                    for rd in rdescs:
                        rd.wait()
                if wdescs[q] is not None:
                    wdescs[q].wait()
                ib = ibufs[q]
                ob = obufs[q]

                def vec_body(j, car):
                    for i in range(16):
                        o = (j * 16 + i) * 16
                        vidx = ib[pl.ds(o, 16)]
                        ob[pl.ds(o, 16)] = plsc.load_gather(row_v, [vidx])
                    return car

                lax.fori_loop(0, CB // 256, vec_body, 0)
                wdescs[q] = pltpu.async_copy(
                    ob, out_hbm.at[c, d, pl.ds(ch * CB, CB)], wsem)
            return carry

        lax.fori_loop(0, NC, col_body, 0)
        drain_two_writes()

    return k(table_t3, idx_t)


def _tc_assemble_body(cat_ref, nv_ref, mf_ref, nf_ref, w1_ref, b1_ref,
                      w2_ref, b2_ref, me_ref, ne_ref, cls_ref, pos_ref,
                      posc_ref, out_ref):
    bb = out_ref.shape[2]
    out_ref[0, :, :] = jnp.broadcast_to(cls_ref[...] + pos_ref[:, 0:1],
                                        (D, bb))
    out_ref[1:1 + NC, :, :] = cat_ref[...] + posc_ref[...]
    for n in range(NN):
        vr = nv_ref[n:n + 1, :]
        mfr = mf_ref[n:n + 1, :]
        nfr = nf_ref[n:n + 1, :]
        sp = jnp.maximum(mfr, nfr)
        v0 = vr * (1.0 - sp)
        h = jnp.maximum(w1_ref[:, n:n + 1] * v0 + b1_ref[:, n:n + 1], 0.0)
        o = jnp.dot(w2_ref[n], h, preferred_element_type=jnp.float32)
        o = o + b2_ref[:, n:n + 1]
        o = jnp.where(mfr > 0.5, me_ref[:, n:n + 1], o)
        o = jnp.where(nfr > 0.5, ne_ref[:, n:n + 1], o)
        out_ref[1 + NC + n, :, :] = o + pos_ref[:, 1 + NC + n:2 + NC + n]


def _tc_assemble(cat_t, nv_t, mf_t, nf_t, w1t, b1t, w2t, b2t, met, net,
                 clst, post, posc3, interpret=False):
    BB = 2048
    grid = (B // BB,)
    return pl.pallas_call(
        _tc_assemble_body,
        grid=grid,
        in_specs=[
            pl.BlockSpec((NC, D, BB), lambda i: (0, 0, i)),
            pl.BlockSpec((NN, BB), lambda i: (0, i)),
            pl.BlockSpec((NN, BB), lambda i: (0, i)),
            pl.BlockSpec((NN, BB), lambda i: (0, i)),
            pl.BlockSpec((D, NN), lambda i: (0, 0)),
            pl.BlockSpec((D, NN), lambda i: (0, 0)),
            pl.BlockSpec((NN, D, D), lambda i: (0, 0, 0)),
            pl.BlockSpec((D, NN), lambda i: (0, 0)),
            pl.BlockSpec((D, NN), lambda i: (0, 0)),
            pl.BlockSpec((D, NN), lambda i: (0, 0)),
            pl.BlockSpec((D, 1), lambda i: (0, 0)),
            pl.BlockSpec((D, NT), lambda i: (0, 0)),
            pl.BlockSpec((NC, D, 1), lambda i: (0, 0, 0)),
        ],
        out_specs=pl.BlockSpec((NT, D, BB), lambda i: (0, 0, i)),
        out_shape=jax.ShapeDtypeStruct((NT, D, B), jnp.float32),
        interpret=interpret,
    )(cat_t, nv_t, mf_t, nf_t, w1t, b1t, w2t, b2t, met, net, clst, post,
      posc3)


def kernel(cat_indices, numeric_values, mask_flags, null_flags, emb_tables,
           W1, b1, W2, b2, mask_emb, null_emb, cls_token, pos_table):
    table_t3 = jnp.transpose(emb_tables, (0, 2, 1))
    idx_t = cat_indices.astype(jnp.int32).T
    cat_t = _sc_gather(table_t3, idx_t)

    nv_t = numeric_values.T
    mf_t = mask_flags.T.astype(jnp.float32)
    nf_t = null_flags.T.astype(jnp.float32)
    w1t = W1.reshape(NN, H).T
    b1t = b1.T
    w2t = jnp.transpose(W2, (0, 2, 1))
    b2t = b2.T
    met = mask_emb.T
    net = null_emb.T
    clst = cls_token.reshape(1, D).T
    post = pos_table.T
    posc3 = pos_table[1:1 + NC][:, :, None]

    out_t = _tc_assemble(cat_t, nv_t, mf_t, nf_t, w1t, b1t, w2t, b2t,
                         met, net, clst, post, posc3)
    return jnp.transpose(out_t, (2, 0, 1))
